# baseline (speedup 1.0000x reference)
import jax
import jax.numpy as jnp
from jax import lax
from jax.experimental import pallas as pl
from jax.experimental.pallas import tpu as pltpu

_P = 8
_K = 2


def _gate_top2_kernel(x_ref, w_ref, g_ref, i_ref):
    xb = x_ref[...]
    w = w_ref[...]
    s = lax.dot_general(w, xb, (((1,), (1,)), ((), ())),
                        preferred_element_type=jnp.float32)
    s = jax.nn.sigmoid(s)
    t = s.shape[1]
    m1 = s[0:1]
    i1 = jnp.zeros((1, t), jnp.int32)
    m2 = jnp.full((1, t), -1.0, jnp.float32)
    i2 = jnp.zeros((1, t), jnp.int32)
    for p in range(1, _P):
        sp = s[p:p + 1]
        pv = jnp.full((1, t), p, jnp.int32)
        b1 = sp > m1
        b2 = sp > m2
        m2 = jnp.where(b1, m1, jnp.where(b2, sp, m2))
        i2 = jnp.where(b1, i1, jnp.where(b2, pv, i2))
        m1 = jnp.where(b1, sp, m1)
        i1 = jnp.where(b1, pv, i1)
    g_ref[...] = jnp.concatenate([m1, m2], axis=0).T
    i_ref[...] = jnp.concatenate([i1, i2], axis=0).T


def kernel(x, W_gate):
    B, L, D = x.shape
    tokens = B * L
    tblk = 2048
    xr = x.reshape(tokens, D)
    grid = (tokens // tblk,)
    g, i = pl.pallas_call(
        _gate_top2_kernel,
        grid=grid,
        in_specs=[
            pl.BlockSpec((tblk, D), lambda t: (t, 0)),
            pl.BlockSpec((_P, D), lambda t: (0, 0)),
        ],
        out_specs=[
            pl.BlockSpec((tblk, _K), lambda t: (t, 0)),
            pl.BlockSpec((tblk, _K), lambda t: (t, 0)),
        ],
        out_shape=[
            jax.ShapeDtypeStruct((tokens, _K), jnp.float32),
            jax.ShapeDtypeStruct((tokens, _K), jnp.int32),
        ],
        compiler_params=pltpu.CompilerParams(
            dimension_semantics=("parallel",),
        ),
    )(xr, W_gate)
    return g.reshape(B, L, _K), i.reshape(B, L, _K)


# speedup vs baseline: 1.7513x; 1.7513x over previous
import jax
import jax.numpy as jnp
from jax import lax
from jax.experimental import pallas as pl
from jax.experimental.pallas import tpu as pltpu

_P = 8
_K = 2


def _gate_top2_kernel(x_ref, w_ref, g_ref, i_ref):
    xb = x_ref[...]
    w = w_ref[...]
    s = lax.dot_general(w, xb, (((1,), (1,)), ((), ())),
                        preferred_element_type=jnp.float32)
    s = jax.nn.sigmoid(s)
    t = s.shape[1]
    m1 = s[0:1]
    i1 = jnp.zeros((1, t), jnp.int32)
    m2 = jnp.full((1, t), -1.0, jnp.float32)
    i2 = jnp.zeros((1, t), jnp.int32)
    for p in range(1, _P):
        sp = s[p:p + 1]
        pv = jnp.full((1, t), p, jnp.int32)
        b1 = sp > m1
        b2 = jnp.logical_and(sp > m2, jnp.logical_not(b1))
        m2 = jnp.where(b1, m1, jnp.where(b2, sp, m2))
        i2 = jnp.where(b1, i1, jnp.where(b2, pv, i2))
        m1 = jnp.where(b1, sp, m1)
        i1 = jnp.where(b1, pv, i1)
    g_ref[...] = jnp.concatenate([m1, m2], axis=0)
    i_ref[...] = jnp.concatenate([i1, i2], axis=0)


def kernel(x, W_gate):
    B, L, D = x.shape
    tokens = B * L
    tblk = 2048
    xr = x.reshape(tokens, D)
    grid = (tokens // tblk,)
    g, i = pl.pallas_call(
        _gate_top2_kernel,
        grid=grid,
        in_specs=[
            pl.BlockSpec((tblk, D), lambda t: (t, 0)),
            pl.BlockSpec((_P, D), lambda t: (0, 0)),
        ],
        out_specs=[
            pl.BlockSpec((_K, tblk), lambda t: (0, t)),
            pl.BlockSpec((_K, tblk), lambda t: (0, t)),
        ],
        out_shape=[
            jax.ShapeDtypeStruct((_K, tokens), jnp.float32),
            jax.ShapeDtypeStruct((_K, tokens), jnp.int32),
        ],
        compiler_params=pltpu.CompilerParams(
            dimension_semantics=("parallel",),
        ),
    )(xr, W_gate)
    g = g.T.reshape(B, L, _K)
    i = i.T.reshape(B, L, _K)
    return g, i
